# baseline (speedup 1.0000x reference)
import jax
import jax.numpy as jnp
from jax.experimental import pallas as pl
from jax.experimental.pallas import tpu as pltpu

_EPS = 1e-5
_BM = 200


def _pass1_kernel(x_ref, w1_ref, adj_a_ref, adj_b_ref, s1_ref, sh1_ref,
                  w2_ref, ta_ref, tb_ref, support_ref):
    @pl.when(pl.program_id(0) == 0)
    def _():
        support_ref[...] = jnp.dot(x_ref[...], w1_ref[...],
                                   preferred_element_type=jnp.float32)

    for adj_ref, t_ref in ((adj_a_ref, ta_ref), (adj_b_ref, tb_ref)):
        acc = jnp.dot(adj_ref[...], support_ref[...],
                      preferred_element_type=jnp.float32)
        h1 = jnp.maximum(acc * s1_ref[...] + sh1_ref[...], 0.0)
        t_ref[...] = jnp.dot(h1, w2_ref[...],
                             preferred_element_type=jnp.float32)


def _pass2_kernel(adj_a_ref, adj_b_ref, ta_ref, tb_ref, s2_ref, sh2_ref,
                  dw_ref, db_ref, mua_ref, mub_ref, outa_ref, outb_ref):
    half = ta_ref.shape[0]
    for adj_ref, mu_ref, out_ref in ((adj_a_ref, mua_ref, outa_ref),
                                     (adj_b_ref, mub_ref, outb_ref)):
        acc = (jnp.dot(adj_ref[:, :half], ta_ref[...],
                       preferred_element_type=jnp.float32)
               + jnp.dot(adj_ref[:, half:], tb_ref[...],
                         preferred_element_type=jnp.float32))
        mu = acc * s2_ref[...] + sh2_ref[...]
        mu_ref[...] = mu
        out_ref[...] = jnp.dot(mu, dw_ref[...],
                               preferred_element_type=jnp.float32) + db_ref[...]


def kernel(x, adj, W1, W2, g1, b1, m1, v1, g2, b2, m2, v2, dec_W, dec_b):
    N, F = x.shape
    H1 = W1.shape[1]
    H2 = W2.shape[1]
    C = dec_W.shape[0]

    inv1 = g1 / jnp.sqrt(v1 + _EPS)
    s1 = inv1.reshape(1, H1)
    sh1 = (b1 - m1 * inv1).reshape(1, H1)
    inv2 = g2 / jnp.sqrt(v2 + _EPS)
    s2 = inv2.reshape(1, H2)
    sh2 = (b2 - m2 * inv2).reshape(1, H2)
    dwT = dec_W.T
    db = dec_b.reshape(1, C)

    BM = _BM
    nblk = N // (2 * BM)
    grid = (nblk,)
    half = N // 2
    const = lambda i: (0, 0)
    top = lambda i: (i, 0)
    bot = lambda i: (i + nblk, 0)

    ta, tb = pl.pallas_call(
        _pass1_kernel,
        grid=grid,
        in_specs=[
            pl.BlockSpec((N, F), const),
            pl.BlockSpec((F, H1), const),
            pl.BlockSpec((BM, N), top),
            pl.BlockSpec((BM, N), bot),
            pl.BlockSpec((1, H1), const),
            pl.BlockSpec((1, H1), const),
            pl.BlockSpec((H1, H2), const),
        ],
        out_specs=[
            pl.BlockSpec((BM, H2), top),
            pl.BlockSpec((BM, H2), top),
        ],
        out_shape=[
            jax.ShapeDtypeStruct((half, H2), jnp.float32),
            jax.ShapeDtypeStruct((half, H2), jnp.float32),
        ],
        scratch_shapes=[pltpu.VMEM((N, H1), jnp.float32)],
    )(x, W1, adj, adj, s1, sh1, W2)

    mua, mub, outa, outb = pl.pallas_call(
        _pass2_kernel,
        grid=grid,
        in_specs=[
            pl.BlockSpec((BM, N), top),
            pl.BlockSpec((BM, N), bot),
            pl.BlockSpec((half, H2), const),
            pl.BlockSpec((half, H2), const),
            pl.BlockSpec((1, H2), const),
            pl.BlockSpec((1, H2), const),
            pl.BlockSpec((H2, C), const),
            pl.BlockSpec((1, C), const),
        ],
        out_specs=[
            pl.BlockSpec((BM, H2), top),
            pl.BlockSpec((BM, H2), top),
            pl.BlockSpec((BM, C), top),
            pl.BlockSpec((BM, C), top),
        ],
        out_shape=[
            jax.ShapeDtypeStruct((half, H2), jnp.float32),
            jax.ShapeDtypeStruct((half, H2), jnp.float32),
            jax.ShapeDtypeStruct((half, C), jnp.float32),
            jax.ShapeDtypeStruct((half, C), jnp.float32),
        ],
    )(adj, adj, ta, tb, s2, sh2, dwT, db)

    mu = jnp.concatenate([mua, mub], axis=0)
    out = jnp.concatenate([outa, outb], axis=0)
    return (out, mu, mu)


# speedup vs baseline: 1.1080x; 1.1080x over previous
import jax
import jax.numpy as jnp
from jax.experimental import pallas as pl
from jax.experimental.pallas import tpu as pltpu

_EPS = 1e-5
_BM = 400


def _pass1_kernel(x_ref, w1_ref, adj_ref, s1_ref, sh1_ref, w2_ref,
                  t_ref, support_ref):
    @pl.when(pl.program_id(0) == 0)
    def _():
        support_ref[...] = jnp.dot(
            x_ref[...].astype(jnp.bfloat16), w1_ref[...],
            preferred_element_type=jnp.float32).astype(jnp.bfloat16)

    acc = jnp.dot(adj_ref[...].astype(jnp.bfloat16), support_ref[...],
                  preferred_element_type=jnp.float32)
    h1 = jnp.maximum(acc * s1_ref[...] + sh1_ref[...], 0.0)
    t_ref[...] = jnp.dot(h1.astype(jnp.bfloat16), w2_ref[...],
                         preferred_element_type=jnp.float32).astype(jnp.bfloat16)


def _pass2_kernel(adj_ref, t_ref, s2_ref, sh2_ref, dw_ref, db_ref,
                  mu_ref, out_ref):
    acc = jnp.dot(adj_ref[...].astype(jnp.bfloat16), t_ref[...],
                  preferred_element_type=jnp.float32)
    mu = acc * s2_ref[...] + sh2_ref[...]
    mu_ref[...] = mu
    out_ref[...] = jnp.dot(mu, dw_ref[...],
                           preferred_element_type=jnp.float32) + db_ref[...]


def kernel(x, adj, W1, W2, g1, b1, m1, v1, g2, b2, m2, v2, dec_W, dec_b):
    N, F = x.shape
    H1 = W1.shape[1]
    H2 = W2.shape[1]
    C = dec_W.shape[0]

    inv1 = g1 / jnp.sqrt(v1 + _EPS)
    s1 = inv1.reshape(1, H1)
    sh1 = (b1 - m1 * inv1).reshape(1, H1)
    inv2 = g2 / jnp.sqrt(v2 + _EPS)
    s2 = inv2.reshape(1, H2)
    sh2 = (b2 - m2 * inv2).reshape(1, H2)
    dwT = dec_W.T
    db = dec_b.reshape(1, C)

    BM = _BM
    grid = (N // BM,)
    const = lambda i: (0, 0)
    row = lambda i: (i, 0)

    t = pl.pallas_call(
        _pass1_kernel,
        grid=grid,
        in_specs=[
            pl.BlockSpec((N, F), const),
            pl.BlockSpec((F, H1), const),
            pl.BlockSpec((BM, N), row),
            pl.BlockSpec((1, H1), const),
            pl.BlockSpec((1, H1), const),
            pl.BlockSpec((H1, H2), const),
        ],
        out_specs=pl.BlockSpec((BM, H2), row),
        out_shape=jax.ShapeDtypeStruct((N, H2), jnp.bfloat16),
        scratch_shapes=[pltpu.VMEM((N, H1), jnp.bfloat16)],
    )(x, W1.astype(jnp.bfloat16), adj, s1, sh1, W2.astype(jnp.bfloat16))

    mu, out = pl.pallas_call(
        _pass2_kernel,
        grid=grid,
        in_specs=[
            pl.BlockSpec((BM, N), row),
            pl.BlockSpec((N, H2), const),
            pl.BlockSpec((1, H2), const),
            pl.BlockSpec((1, H2), const),
            pl.BlockSpec((H2, C), const),
            pl.BlockSpec((1, C), const),
        ],
        out_specs=[
            pl.BlockSpec((BM, H2), row),
            pl.BlockSpec((BM, C), row),
        ],
        out_shape=[
            jax.ShapeDtypeStruct((N, H2), jnp.float32),
            jax.ShapeDtypeStruct((N, C), jnp.float32),
        ],
    )(adj, t, s2, sh2, dwT, db)

    return (out, mu, mu)
